# baseline (speedup 1.0000x reference)
import functools
import jax
import jax.numpy as jnp
from jax import lax
from jax.experimental import pallas as pl
from jax.experimental.pallas import tpu as pltpu
from jax.experimental.pallas import tpu_sc as plsc

_N = 100000
_D = 128
_NTC = 80000
_CH = 40000
_NCH = _NTC // _CH

_NSC = _N - _NTC
_NC = 2
_NS = 16
_NW = _NC * _NS
_SCCH = 200
_SCG = _NSC // _SCCH
_SCVECS = _SCCH * _D // 16


def _tc_body(p_hbm, z_hbm, buf, lsem, ssem):
    def load(c):
        pltpu.make_async_copy(
            p_hbm.at[pl.ds(c * _CH, _CH)], buf.at[c], lsem.at[c]
        ).start()

    def wait_load(c):
        pltpu.make_async_copy(
            p_hbm.at[pl.ds(c * _CH, _CH)], buf.at[c], lsem.at[c]
        ).wait()

    def store(c):
        pltpu.make_async_copy(
            buf.at[c], z_hbm.at[pl.ds(c * _CH, _CH)], ssem.at[c]
        ).start()

    def wait_store(c):
        pltpu.make_async_copy(
            buf.at[c], z_hbm.at[pl.ds(c * _CH, _CH)], ssem.at[c]
        ).wait()

    load(0)
    load(1)
    for c in range(_NCH):
        wait_load(c)
        buf[c] = 0.5 * jnp.tanh(buf[c] * 0.5) + 0.5
        store(c)
    wait_store(0)
    wait_store(1)


def _tc_part(P_head):
    return pl.pallas_call(
        _tc_body,
        in_specs=[pl.BlockSpec(memory_space=pl.ANY)],
        out_specs=pl.BlockSpec(memory_space=pl.ANY),
        out_shape=jax.ShapeDtypeStruct((_NTC, _D), jnp.float32),
        scratch_shapes=[
            pltpu.VMEM((_NCH, _CH, _D), jnp.float32),
            pltpu.SemaphoreType.DMA((_NCH,)),
            pltpu.SemaphoreType.DMA((_NCH,)),
        ],
    )(P_head)


_mesh = plsc.VectorSubcoreMesh(core_axis_name="c", subcore_axis_name="s")


@functools.partial(
    pl.kernel,
    mesh=_mesh,
    out_type=jax.ShapeDtypeStruct((_NSC, _D), jnp.float32),
    scratch_types=[
        pltpu.VMEM((_SCCH, _D), jnp.float32),
        pltpu.VMEM((_SCCH, _D), jnp.float32),
    ],
)
def _sc_part(p_hbm, z_hbm, inb, outb):
    wid = lax.axis_index("s") * _NC + lax.axis_index("c")
    trip = (_SCG - wid + _NW - 1) // _NW

    def chunk_body(t, _):
        g = wid + t * _NW
        row0 = pl.multiple_of(g * _SCCH, 8)
        pltpu.sync_copy(p_hbm.at[pl.ds(row0, _SCCH)], inb)

        def vec_body(j, _):
            r = j // 8
            k = (j % 8) * 16
            x = inb[r, pl.ds(k, 16)]
            outb[r, pl.ds(k, 16)] = 1.0 / (1.0 + jnp.exp(-x))
            return 0

        lax.fori_loop(0, _SCVECS, vec_body, 0)
        pltpu.sync_copy(outb, z_hbm.at[pl.ds(row0, _SCCH)])
        return 0

    lax.fori_loop(0, trip, chunk_body, 0)


def kernel(P, test):
    z_tc = _tc_part(P[:_NTC])
    z_sc = _sc_part(P[_NTC:])
    return jnp.concatenate([z_tc, z_sc], axis=0)


# speedup vs baseline: 3.7442x; 3.7442x over previous
import jax
import jax.numpy as jnp
from jax.experimental import pallas as pl
from jax.experimental.pallas import tpu as pltpu

_N = 100000
_D = 128
_CH = 50000
_NCH = _N // _CH


def _body(p_hbm, z_hbm, buf, lsem, ssem):
    def load(c):
        pltpu.make_async_copy(
            p_hbm.at[pl.ds(c * _CH, _CH)], buf.at[c], lsem.at[c]
        ).start()

    def wait_load(c):
        pltpu.make_async_copy(
            p_hbm.at[pl.ds(c * _CH, _CH)], buf.at[c], lsem.at[c]
        ).wait()

    def store(c):
        pltpu.make_async_copy(
            buf.at[c], z_hbm.at[pl.ds(c * _CH, _CH)], ssem.at[c]
        ).start()

    def wait_store(c):
        pltpu.make_async_copy(
            buf.at[c], z_hbm.at[pl.ds(c * _CH, _CH)], ssem.at[c]
        ).wait()

    load(0)
    load(1)
    for c in range(_NCH):
        wait_load(c)
        buf[c] = 0.5 * jnp.tanh(buf[c] * 0.5) + 0.5
        store(c)
    wait_store(0)
    wait_store(1)


def kernel(P, test):
    return pl.pallas_call(
        _body,
        in_specs=[pl.BlockSpec(memory_space=pl.ANY)],
        out_specs=pl.BlockSpec(memory_space=pl.ANY),
        out_shape=jax.ShapeDtypeStruct((_N, _D), jnp.float32),
        scratch_shapes=[
            pltpu.VMEM((_NCH, _CH, _D), jnp.float32),
            pltpu.SemaphoreType.DMA((_NCH,)),
            pltpu.SemaphoreType.DMA((_NCH,)),
        ],
    )(P)
